# Initial kernel scaffold; baseline (speedup 1.0000x reference)
import jax
import jax.numpy as jnp
from jax.experimental import pallas as pl


def kernel(seq, W, K):
    raise NotImplementedError("write your pallas kernel here")


# speedup vs baseline: 7.0873x; 7.0873x over previous
import functools

import jax
import jax.numpy as jnp
from jax import lax
from jax.experimental import pallas as pl
from jax.experimental.pallas import tpu as pltpu
from jax.experimental.pallas import tpu_sc as plsc

_VOCAB = 100000
_EMB = 64
_RMAX = 7
_B = 1024
_L = 200
_NTOK = _B * _L

_NC, _NS = 2, 16
_NW = _NC * _NS
_TOK_PER_W = _NTOK // _NW
_CHUNK = 80
_NBUF = 2
_NCHUNK = _TOK_PER_W // _CHUNK
_NROUND = _NCHUNK // _NBUF

_KROW = _RMAX * _EMB


def _sc_gather(seq_flat, W, K2):
    mesh = plsc.VectorSubcoreMesh(core_axis_name="c", subcore_axis_name="s")

    @functools.partial(
        pl.kernel,
        out_type=(
            jax.ShapeDtypeStruct((_NTOK, _EMB), jnp.float32),
            jax.ShapeDtypeStruct((_NTOK, _KROW), jnp.float32),
        ),
        mesh=mesh,
        scratch_types=[
            pltpu.VMEM((_NBUF, _CHUNK), jnp.int32),
            pltpu.VMEM((_NBUF, _CHUNK, _EMB), jnp.float32),
            pltpu.VMEM((_NBUF, _CHUNK, _KROW), jnp.float32),
            pltpu.SemaphoreType.DMA,
            pltpu.SemaphoreType.DMA,
            pltpu.SemaphoreType.DMA,
            pltpu.SemaphoreType.DMA,
        ],
        compiler_params=pltpu.CompilerParams(use_tc_tiling_on_sc=False),
    )
    def gather_kernel(seq_hbm, w_hbm, k2_hbm, wg_hbm, kg_hbm,
                      idx_v, wrow_v, krow_v, gsem0, gsem1, wsem0, wsem1):
        wid = lax.axis_index("s") * _NC + lax.axis_index("c")
        base = wid * _TOK_PER_W
        gsems = (gsem0, gsem1)
        wsems = (wsem0, wsem1)

        def wait_writeback(b):
            pltpu.make_async_copy(
                wrow_v.at[b], wg_hbm.at[pl.ds(base, _CHUNK)], wsems[b]).wait()
            pltpu.make_async_copy(
                krow_v.at[b], kg_hbm.at[pl.ds(base, _CHUNK)], wsems[b]).wait()

        @pl.loop(0, _NROUND)
        def _round(g):
            @pl.when(g > 0)
            def _():
                wait_writeback(0)
                wait_writeback(1)

            copies = []
            for b in range(_NBUF):
                off = base + (g * _NBUF + b) * _CHUNK
                pltpu.sync_copy(seq_hbm.at[pl.ds(off, _CHUNK)], idx_v.at[b])
                cw = pltpu.async_copy(w_hbm.at[idx_v.at[b]], wrow_v.at[b],
                                      gsems[b])
                ck = pltpu.async_copy(k2_hbm.at[idx_v.at[b]], krow_v.at[b],
                                      gsems[b])
                copies.append((cw, ck))
            for b in range(_NBUF):
                cw, ck = copies[b]
                cw.wait()
                ck.wait()
                off = base + (g * _NBUF + b) * _CHUNK
                pltpu.async_copy(wrow_v.at[b], wg_hbm.at[pl.ds(off, _CHUNK)],
                                 wsems[b])
                pltpu.async_copy(krow_v.at[b], kg_hbm.at[pl.ds(off, _CHUNK)],
                                 wsems[b])

        wait_writeback(0)
        wait_writeback(1)

    return gather_kernel(seq_flat, W, K2)


def _tc_merge(Wg, Kg):
    bb = 8
    n3, n5, n7 = _L - 2, _L - 4, _L - 6

    def body(wg_ref, kg_ref, o3_ref, o5_ref, o7_ref):
        for b in range(bb):
            def prod(d, clo, n):
                w = wg_ref[b, pl.ds(clo + d, n), :]
                k = kg_ref[b, pl.ds(clo, n), pl.ds(_EMB * (3 + d), _EMB)]
                return w * k

            m = prod(-1, 1, n3)
            m = jnp.maximum(m, prod(0, 1, n3))
            m = jnp.maximum(m, prod(1, 1, n3))
            o3_ref[b] = m
            m = m[1:1 + n5]
            m = jnp.maximum(m, prod(-2, 2, n5))
            m = jnp.maximum(m, prod(2, 2, n5))
            o5_ref[b] = m
            m = m[1:1 + n7]
            m = jnp.maximum(m, prod(-3, 3, n7))
            m = jnp.maximum(m, prod(3, 3, n7))
            o7_ref[b] = m

    out = pl.pallas_call(
        body,
        grid=(_B // bb,),
        in_specs=[
            pl.BlockSpec((bb, _L, _EMB), lambda i: (i, 0, 0)),
            pl.BlockSpec((bb, _L, _KROW), lambda i: (i, 0, 0)),
        ],
        out_specs=[
            pl.BlockSpec((bb, n3, _EMB), lambda i: (i, 0, 0)),
            pl.BlockSpec((bb, n5, _EMB), lambda i: (i, 0, 0)),
            pl.BlockSpec((bb, n7, _EMB), lambda i: (i, 0, 0)),
        ],
        out_shape=[
            jax.ShapeDtypeStruct((_B, n3, _EMB), jnp.float32),
            jax.ShapeDtypeStruct((_B, n5, _EMB), jnp.float32),
            jax.ShapeDtypeStruct((_B, n7, _EMB), jnp.float32),
        ],
    )(Wg, Kg)
    return tuple(out)


@jax.jit
def kernel(seq, W, K):
    seq_flat = seq.astype(jnp.int32).reshape(-1)
    K2 = K.reshape(_VOCAB, _KROW)
    Wg, Kg = _sc_gather(seq_flat, W, K2)
    Wg = Wg.reshape(_B, _L, _EMB)
    Kg = Kg.reshape(_B, _L, _KROW)
    return _tc_merge(Wg, Kg)


# speedup vs baseline: 8.9536x; 1.2633x over previous
import functools

import jax
import jax.numpy as jnp
from jax import lax
from jax.experimental import pallas as pl
from jax.experimental.pallas import tpu as pltpu
from jax.experimental.pallas import tpu_sc as plsc

_VOCAB = 100000
_EMB = 64
_RMAX = 7
_B = 1024
_L = 200
_NTOK = _B * _L

_WPAD = 128
_KPAD = 512

_NC, _NS = 2, 16
_NW = _NC * _NS
_TOK_PER_W = _NTOK // _NW
_CHUNK = 64
_NBUF = 2
_NCHUNK = _TOK_PER_W // _CHUNK
_NROUND = _NCHUNK // _NBUF


def _sc_gather(seq_flat, Wp, Kp):
    mesh = plsc.VectorSubcoreMesh(core_axis_name="c", subcore_axis_name="s")

    @functools.partial(
        pl.kernel,
        out_type=(
            jax.ShapeDtypeStruct((_NTOK, _WPAD), jnp.float32),
            jax.ShapeDtypeStruct((_NTOK, _KPAD), jnp.float32),
        ),
        mesh=mesh,
        scratch_types=[
            pltpu.VMEM((_NBUF, _CHUNK), jnp.int32),
            pltpu.VMEM((_NBUF, _CHUNK, _WPAD), jnp.float32),
            pltpu.VMEM((_NBUF, _CHUNK, _KPAD), jnp.float32),
            pltpu.SemaphoreType.DMA,
            pltpu.SemaphoreType.DMA,
            pltpu.SemaphoreType.DMA,
            pltpu.SemaphoreType.DMA,
        ],
    )
    def gather_kernel(seq_hbm, w_hbm, k_hbm, wg_hbm, kg_hbm,
                      idx_v, wrow_v, krow_v, gsem0, gsem1, wsem0, wsem1):
        wid = lax.axis_index("s") * _NC + lax.axis_index("c")
        base = wid * _TOK_PER_W
        gsems = (gsem0, gsem1)
        wsems = (wsem0, wsem1)

        def wait_writeback(b):
            pltpu.make_async_copy(
                wrow_v.at[b], wg_hbm.at[pl.ds(base, _CHUNK)], wsems[b]).wait()
            pltpu.make_async_copy(
                krow_v.at[b], kg_hbm.at[pl.ds(base, _CHUNK)], wsems[b]).wait()

        @pl.loop(0, _NROUND)
        def _round(g):
            @pl.when(g > 0)
            def _():
                wait_writeback(0)
                wait_writeback(1)

            copies = []
            for b in range(_NBUF):
                off = base + (g * _NBUF + b) * _CHUNK
                pltpu.sync_copy(seq_hbm.at[pl.ds(off, _CHUNK)], idx_v.at[b])
                cw = pltpu.async_copy(w_hbm.at[idx_v.at[b]], wrow_v.at[b],
                                      gsems[b])
                ck = pltpu.async_copy(k_hbm.at[idx_v.at[b]], krow_v.at[b],
                                      gsems[b])
                copies.append((cw, ck))
            for b in range(_NBUF):
                cw, ck = copies[b]
                cw.wait()
                ck.wait()
                off = base + (g * _NBUF + b) * _CHUNK
                pltpu.async_copy(wrow_v.at[b], wg_hbm.at[pl.ds(off, _CHUNK)],
                                 wsems[b])
                pltpu.async_copy(krow_v.at[b], kg_hbm.at[pl.ds(off, _CHUNK)],
                                 wsems[b])

        wait_writeback(0)
        wait_writeback(1)

    return gather_kernel(seq_flat, Wp, Kp)


def _tc_merge(Wg, Kg):
    bb = 8
    n3, n5, n7 = _L - 2, _L - 4, _L - 6

    def body(wg_ref, kg_ref, o3_ref, o5_ref, o7_ref):
        for b in range(bb):
            def prod(d, clo, n):
                w = wg_ref[b, pl.ds(clo + d, n), pl.ds(0, _EMB)]
                k = kg_ref[b, pl.ds(clo, n), pl.ds(_EMB * (3 + d), _EMB)]
                return w * k

            m = prod(-1, 1, n3)
            m = jnp.maximum(m, prod(0, 1, n3))
            m = jnp.maximum(m, prod(1, 1, n3))
            o3_ref[b] = m
            m = m[1:1 + n5]
            m = jnp.maximum(m, prod(-2, 2, n5))
            m = jnp.maximum(m, prod(2, 2, n5))
            o5_ref[b] = m
            m = m[1:1 + n7]
            m = jnp.maximum(m, prod(-3, 3, n7))
            m = jnp.maximum(m, prod(3, 3, n7))
            o7_ref[b] = m

    out = pl.pallas_call(
        body,
        grid=(_B // bb,),
        in_specs=[
            pl.BlockSpec((bb, _L, _WPAD), lambda i: (i, 0, 0)),
            pl.BlockSpec((bb, _L, _KPAD), lambda i: (i, 0, 0)),
        ],
        out_specs=[
            pl.BlockSpec((bb, n3, _EMB), lambda i: (i, 0, 0)),
            pl.BlockSpec((bb, n5, _EMB), lambda i: (i, 0, 0)),
            pl.BlockSpec((bb, n7, _EMB), lambda i: (i, 0, 0)),
        ],
        out_shape=[
            jax.ShapeDtypeStruct((_B, n3, _EMB), jnp.float32),
            jax.ShapeDtypeStruct((_B, n5, _EMB), jnp.float32),
            jax.ShapeDtypeStruct((_B, n7, _EMB), jnp.float32),
        ],
    )(Wg, Kg)
    return tuple(out)


@jax.jit
def kernel(seq, W, K):
    seq_flat = seq.astype(jnp.int32).reshape(-1)
    Wp = jnp.pad(W, ((0, 0), (0, _WPAD - _EMB)))
    Kp = jnp.pad(K.reshape(_VOCAB, _RMAX * _EMB),
                 ((0, 0), (0, _KPAD - _RMAX * _EMB)))
    Wg, Kg = _sc_gather(seq_flat, Wp, Kp)
    Wg = Wg.reshape(_B, _L, _WPAD)
    Kg = Kg.reshape(_B, _L, _KPAD)
    return _tc_merge(Wg, Kg)
